# Initial kernel scaffold; baseline (speedup 1.0000x reference)
import jax
import jax.numpy as jnp
from jax.experimental import pallas as pl


def kernel(seq, embed, ff_w1, ff_b1, ff_w2, ff_b2, ln_g, ln_b, sem_w, sem_b, epi_w, epi_b, rtr_w, rtr_b, out_w, out_b):
    raise NotImplementedError("write your pallas kernel here")


# speedup vs baseline: 19.9199x; 19.9199x over previous
import functools

import jax
import jax.numpy as jnp
from jax.experimental import pallas as pl
from jax.experimental.pallas import tpu as pltpu

H = 64
V = 64
HALF = 32
B, L = 256, 2048

TSUB = 8
ENC_N = TSUB * B
TC = 128
NTC = L // TC


def _encoder_body(seq_ref, embT_ref, w1T_ref, b1_ref, w2T_ref, b2_ref,
                  lng_ref, lnb_ref, semT_ref, semb_ref, epiT_ref, epib_ref,
                  rtrT_ref, rtrb_ref, ks_ref, ke_ref, g_ref):
    seq_row = seq_ref[0]
    iota_v = jax.lax.broadcasted_iota(jnp.int32, (V, ENC_N), 0)
    onehot = jnp.where(iota_v == seq_row, 1.0, 0.0)
    e = jnp.dot(embT_ref[...], onehot, preferred_element_type=jnp.float32)

    a1 = jnp.maximum(
        jnp.dot(w1T_ref[...], e, preferred_element_type=jnp.float32)
        + b1_ref[...], 0.0)
    f = jnp.dot(w2T_ref[...], a1, preferred_element_type=jnp.float32) + b2_ref[...]
    x = e + f
    mu = jnp.mean(x, axis=0, keepdims=True)
    xc = x - mu
    var = jnp.mean(xc * xc, axis=0, keepdims=True)
    h = xc * jax.lax.rsqrt(var + 1e-5) * lng_ref[...] + lnb_ref[...]

    ks = jnp.dot(semT_ref[...], h, preferred_element_type=jnp.float32) + semb_ref[...]
    ke = jnp.dot(epiT_ref[...], h, preferred_element_type=jnp.float32) + epib_ref[...]
    gr = jax.nn.sigmoid(
        jnp.dot(rtrT_ref[...], h, preferred_element_type=jnp.float32) + rtrb_ref[...])

    for s in range(TSUB):
        sl = slice(s * B, (s + 1) * B)
        ks_ref[s] = ks[:, sl]
        ke_ref[s] = ke[:, sl]
        g_ref[s] = gr[:, sl]


def _scan_body(ks_ref, ke_ref, g_ref, outwT_ref, outb_ref, out_ref,
               ms_ref, me_ref):
    t = pl.program_id(1)

    @pl.when(t == 0)
    def _():
        ms_ref[...] = jnp.zeros_like(ms_ref)
        me_ref[...] = jnp.zeros_like(me_ref)

    n_steps = jnp.where(t == NTC - 1, TC - 1, TC)
    t0 = (t * TC).astype(jnp.float32)

    def body(s, carry):
        Ms, Me = carry
        k = ks_ref[s]
        q = ke_ref[s]
        g = g_ref[s]
        r = (t0 + s.astype(jnp.float32) + 1.0) * (1.0 / L)

        vps = jnp.sum(Ms * k[:, None, :], axis=0)
        ns = jnp.sum(k * k, axis=0, keepdims=True) + 1e-6
        dvs = k - vps / ns
        Ms = Ms + k[:, None, :] * (g * dvs)[None, :, :]

        vpe = jnp.sum(Me * q[:, None, :], axis=0)
        ne = jnp.sum(q * q, axis=0, keepdims=True) + 1e-6
        dve = q - vpe / ne
        Me = Me + q[:, None, :] * (((1.0 - g) * r) * dve)[None, :, :]
        return Ms, Me

    Ms, Me = jax.lax.fori_loop(0, n_steps, body, (ms_ref[...], me_ref[...]))
    ms_ref[...] = Ms
    me_ref[...] = Me

    @pl.when(t == NTC - 1)
    def _():
        qs = ks_ref[TC - 1]
        qe = ke_ref[TC - 1]
        cs = jnp.sum(Ms * qs[:, None, :], axis=0)
        ce = jnp.sum(Me * qe[:, None, :], axis=0)
        cat = jnp.concatenate([cs, ce], axis=0)
        outT = jnp.dot(outwT_ref[...], cat, preferred_element_type=jnp.float32)
        out_ref[...] = outT.T + outb_ref[...]


def kernel(seq, embed, ff_w1, ff_b1, ff_w2, ff_b2, ln_g, ln_b,
           sem_w, sem_b, epi_w, epi_b, rtr_w, rtr_b, out_w, out_b):
    f32 = jnp.float32
    seq3 = jnp.transpose(seq).reshape(L // TSUB, 1, ENC_N)
    col = lambda v: v.reshape(-1, 1).astype(f32)

    n_enc = L // TSUB
    wspec = lambda shape: pl.BlockSpec(shape, lambda c, i: (0, 0))
    ksT, keT, gT = pl.pallas_call(
        _encoder_body,
        grid=(2, n_enc // 2),
        in_specs=[
            pl.BlockSpec((1, 1, ENC_N), lambda c, i: (c * (n_enc // 2) + i, 0, 0)),
            wspec((H, V)), wspec((2 * H, H)), wspec((2 * H, 1)),
            wspec((H, 2 * H)), wspec((H, 1)), wspec((H, 1)), wspec((H, 1)),
            wspec((HALF, H)), wspec((HALF, 1)), wspec((HALF, H)), wspec((HALF, 1)),
            wspec((1, H)), wspec((1, 1)),
        ],
        out_specs=[
            pl.BlockSpec((TSUB, HALF, B), lambda c, i: (c * (n_enc // 2) + i, 0, 0)),
            pl.BlockSpec((TSUB, HALF, B), lambda c, i: (c * (n_enc // 2) + i, 0, 0)),
            pl.BlockSpec((TSUB, 1, B), lambda c, i: (c * (n_enc // 2) + i, 0, 0)),
        ],
        out_shape=[
            jax.ShapeDtypeStruct((L, HALF, B), f32),
            jax.ShapeDtypeStruct((L, HALF, B), f32),
            jax.ShapeDtypeStruct((L, 1, B), f32),
        ],
        compiler_params=pltpu.CompilerParams(
            dimension_semantics=("parallel", "arbitrary"),
        ),
        name="router_encoder",
    )(seq3, jnp.transpose(embed), jnp.transpose(ff_w1), col(ff_b1),
      jnp.transpose(ff_w2), col(ff_b2), col(ln_g), col(ln_b),
      jnp.transpose(sem_w), col(sem_b), jnp.transpose(epi_w), col(epi_b),
      jnp.transpose(rtr_w), rtr_b.reshape(1, 1))

    BB = B // 2
    cspec = lambda shape: pl.BlockSpec(shape, lambda c, t: (0, 0))
    out = pl.pallas_call(
        _scan_body,
        grid=(2, NTC),
        in_specs=[
            pl.BlockSpec((TC, HALF, BB), lambda c, t: (t, 0, c)),
            pl.BlockSpec((TC, HALF, BB), lambda c, t: (t, 0, c)),
            pl.BlockSpec((TC, 1, BB), lambda c, t: (t, 0, c)),
            cspec((V, 2 * HALF)), cspec((1, V)),
        ],
        out_specs=pl.BlockSpec((BB, V), lambda c, t: (c, 0)),
        out_shape=jax.ShapeDtypeStruct((B, V), f32),
        scratch_shapes=[
            pltpu.VMEM((HALF, HALF, BB), f32),
            pltpu.VMEM((HALF, HALF, BB), f32),
        ],
        compiler_params=pltpu.CompilerParams(
            dimension_semantics=("parallel", "arbitrary"),
        ),
        name="router_scan",
    )(ksT, keT, gT, jnp.transpose(out_w), out_b.reshape(1, V))
    return out


# speedup vs baseline: 30.2980x; 1.5210x over previous
import functools

import jax
import jax.numpy as jnp
from jax.experimental import pallas as pl
from jax.experimental.pallas import tpu as pltpu

H = 64
V = 64
HALF = 32
B, L = 256, 2048

TSUB = 8
ENC_N = TSUB * B
TC = 128
NTC = L // TC


def _encoder_body(seq_ref, embT_ref, w1T_ref, b1_ref, w2T_ref, b2_ref,
                  lng_ref, lnb_ref, semT_ref, semb_ref, epiT_ref, epib_ref,
                  rtrT_ref, rtrb_ref, ks_ref, ke_ref, g_ref):
    seq_row = seq_ref[0]
    iota_v = jax.lax.broadcasted_iota(jnp.int32, (V, ENC_N), 0)
    onehot = jnp.where(iota_v == seq_row, 1.0, 0.0)
    e = jnp.dot(embT_ref[...], onehot, preferred_element_type=jnp.float32)

    a1 = jnp.maximum(
        jnp.dot(w1T_ref[...], e, preferred_element_type=jnp.float32)
        + b1_ref[...], 0.0)
    f = jnp.dot(w2T_ref[...], a1, preferred_element_type=jnp.float32) + b2_ref[...]
    x = e + f
    mu = jnp.mean(x, axis=0, keepdims=True)
    xc = x - mu
    var = jnp.mean(xc * xc, axis=0, keepdims=True)
    h = xc * jax.lax.rsqrt(var + 1e-5) * lng_ref[...] + lnb_ref[...]

    ks = jnp.dot(semT_ref[...], h, preferred_element_type=jnp.float32) + semb_ref[...]
    ke = jnp.dot(epiT_ref[...], h, preferred_element_type=jnp.float32) + epib_ref[...]
    gr = jax.nn.sigmoid(
        jnp.dot(rtrT_ref[...], h, preferred_element_type=jnp.float32) + rtrb_ref[...])

    for s in range(TSUB):
        sl = slice(s * B, (s + 1) * B)
        ks_ref[s] = ks[:, sl]
        ke_ref[s] = ke[:, sl]
        g_ref[s] = gr[:, sl]


def _scan_body(ks_ref, ke_ref, g_ref, outwT_ref, outb_ref, out_ref,
               ms_ref, me_ref):
    t = pl.program_id(1)

    @pl.when(t == 0)
    def _():
        ms_ref[...] = jnp.zeros_like(ms_ref)
        me_ref[...] = jnp.zeros_like(me_ref)

    n_steps = jnp.where(t == NTC - 1, TC - 1, TC)
    t0 = (t * TC).astype(jnp.float32)

    def matvec(m_ref, krows):
        acc = m_ref[0] * krows[0]
        for j in range(1, HALF):
            acc = acc + m_ref[j] * krows[j]
        return acc

    def update(m_ref, krows, gd):
        for j in range(HALF):
            m_ref[j] = m_ref[j] + krows[j] * gd

    def body(s, carry):
        k = ks_ref[s]
        q = ke_ref[s]
        g = g_ref[s]
        r = (t0 + s.astype(jnp.float32) + 1.0) * (1.0 / L)
        krows = [k[j:j + 1, :] for j in range(HALF)]
        qrows = [q[j:j + 1, :] for j in range(HALF)]

        vps = matvec(ms_ref, krows)
        ns = jnp.sum(k * k, axis=0, keepdims=True) + 1e-6
        dvs = k - vps / ns
        update(ms_ref, krows, g * dvs)

        vpe = matvec(me_ref, qrows)
        ne = jnp.sum(q * q, axis=0, keepdims=True) + 1e-6
        dve = q - vpe / ne
        update(me_ref, qrows, ((1.0 - g) * r) * dve)
        return carry

    jax.lax.fori_loop(0, n_steps, body, 0, unroll=False)

    @pl.when(t == NTC - 1)
    def _():
        qs = ks_ref[TC - 1]
        qe = ke_ref[TC - 1]
        cs = matvec(ms_ref, [qs[j:j + 1, :] for j in range(HALF)])
        ce = matvec(me_ref, [qe[j:j + 1, :] for j in range(HALF)])
        cat = jnp.concatenate([cs, ce], axis=0)
        outT = jnp.dot(outwT_ref[...], cat, preferred_element_type=jnp.float32)
        out_ref[...] = outT.T + outb_ref[...]


def kernel(seq, embed, ff_w1, ff_b1, ff_w2, ff_b2, ln_g, ln_b,
           sem_w, sem_b, epi_w, epi_b, rtr_w, rtr_b, out_w, out_b):
    f32 = jnp.float32
    seq3 = jnp.transpose(seq).reshape(L // TSUB, 1, ENC_N)
    col = lambda v: v.reshape(-1, 1).astype(f32)

    n_enc = L // TSUB
    wspec = lambda shape: pl.BlockSpec(shape, lambda c, i: (0, 0))
    ksT, keT, gT = pl.pallas_call(
        _encoder_body,
        grid=(2, n_enc // 2),
        in_specs=[
            pl.BlockSpec((1, 1, ENC_N), lambda c, i: (c * (n_enc // 2) + i, 0, 0)),
            wspec((H, V)), wspec((2 * H, H)), wspec((2 * H, 1)),
            wspec((H, 2 * H)), wspec((H, 1)), wspec((H, 1)), wspec((H, 1)),
            wspec((HALF, H)), wspec((HALF, 1)), wspec((HALF, H)), wspec((HALF, 1)),
            wspec((1, H)), wspec((1, 1)),
        ],
        out_specs=[
            pl.BlockSpec((TSUB, HALF, B), lambda c, i: (c * (n_enc // 2) + i, 0, 0)),
            pl.BlockSpec((TSUB, HALF, B), lambda c, i: (c * (n_enc // 2) + i, 0, 0)),
            pl.BlockSpec((TSUB, 1, B), lambda c, i: (c * (n_enc // 2) + i, 0, 0)),
        ],
        out_shape=[
            jax.ShapeDtypeStruct((L, HALF, B), f32),
            jax.ShapeDtypeStruct((L, HALF, B), f32),
            jax.ShapeDtypeStruct((L, 1, B), f32),
        ],
        compiler_params=pltpu.CompilerParams(
            dimension_semantics=("parallel", "arbitrary"),
        ),
        name="router_encoder",
    )(seq3, jnp.transpose(embed), jnp.transpose(ff_w1), col(ff_b1),
      jnp.transpose(ff_w2), col(ff_b2), col(ln_g), col(ln_b),
      jnp.transpose(sem_w), col(sem_b), jnp.transpose(epi_w), col(epi_b),
      jnp.transpose(rtr_w), rtr_b.reshape(1, 1))

    BB = B // 2
    cspec = lambda shape: pl.BlockSpec(shape, lambda c, t: (0, 0))
    out = pl.pallas_call(
        _scan_body,
        grid=(2, NTC),
        in_specs=[
            pl.BlockSpec((TC, HALF, BB), lambda c, t: (t, 0, c)),
            pl.BlockSpec((TC, HALF, BB), lambda c, t: (t, 0, c)),
            pl.BlockSpec((TC, 1, BB), lambda c, t: (t, 0, c)),
            cspec((V, 2 * HALF)), cspec((1, V)),
        ],
        out_specs=pl.BlockSpec((BB, V), lambda c, t: (c, 0)),
        out_shape=jax.ShapeDtypeStruct((B, V), f32),
        scratch_shapes=[
            pltpu.VMEM((HALF, HALF, BB), f32),
            pltpu.VMEM((HALF, HALF, BB), f32),
        ],
        compiler_params=pltpu.CompilerParams(
            dimension_semantics=("parallel", "arbitrary"),
        ),
        name="router_scan",
    )(ksT, keT, gT, jnp.transpose(out_w), out_b.reshape(1, V))
    return out


# speedup vs baseline: 32.9897x; 1.0888x over previous
import functools

import jax
import jax.numpy as jnp
from jax.experimental import pallas as pl
from jax.experimental.pallas import tpu as pltpu

H = 64
V = 64
HALF = 32
B, L = 256, 2048

TSUB = 8
ENC_N = TSUB * B
TC = 128
NTC = L // TC


def _encoder_body(seq_ref, embT_ref, w1T_ref, b1_ref, w2T_ref, b2_ref,
                  lng_ref, lnb_ref, semT_ref, semb_ref, epiT_ref, epib_ref,
                  rtrT_ref, rtrb_ref, ks_ref, ke_ref, g_ref):
    seq_row = seq_ref[0]
    iota_v = jax.lax.broadcasted_iota(jnp.int32, (V, ENC_N), 0)
    onehot = jnp.where(iota_v == seq_row, 1.0, 0.0)
    e = jnp.dot(embT_ref[...], onehot, preferred_element_type=jnp.float32)

    a1 = jnp.maximum(
        jnp.dot(w1T_ref[...], e, preferred_element_type=jnp.float32)
        + b1_ref[...], 0.0)
    f = jnp.dot(w2T_ref[...], a1, preferred_element_type=jnp.float32) + b2_ref[...]
    x = e + f
    mu = jnp.mean(x, axis=0, keepdims=True)
    xc = x - mu
    var = jnp.mean(xc * xc, axis=0, keepdims=True)
    h = xc * jax.lax.rsqrt(var + 1e-5) * lng_ref[...] + lnb_ref[...]

    ks = jnp.dot(semT_ref[...], h, preferred_element_type=jnp.float32) + semb_ref[...]
    ke = jnp.dot(epiT_ref[...], h, preferred_element_type=jnp.float32) + epib_ref[...]
    gr = jax.nn.sigmoid(
        jnp.dot(rtrT_ref[...], h, preferred_element_type=jnp.float32) + rtrb_ref[...])

    for s in range(TSUB):
        sl = slice(s * B, (s + 1) * B)
        ks_ref[s] = ks[:, sl]
        ke_ref[s] = ke[:, sl]
        g_ref[s] = gr[:, sl]


def _scan_body(ks_ref, ke_ref, g_ref, outwT_ref, outb_ref, out_ref,
               ms_ref, me_ref):
    t = pl.program_id(1)

    @pl.when(t == 0)
    def _():
        ms_ref[...] = jnp.zeros_like(ms_ref)
        me_ref[...] = jnp.zeros_like(me_ref)

    n_steps = jnp.where(t == NTC - 1, TC - 1, TC)
    t0 = (t * TC).astype(jnp.float32)

    def rows(k):
        return [k[j:j + 1, :] for j in range(HALF)]

    def matvec(m_ref, krows):
        acc = [m_ref[j] * krows[j] for j in range(4)]
        for j in range(4, HALF):
            acc[j % 4] = acc[j % 4] + m_ref[j] * krows[j]
        return (acc[0] + acc[1]) + (acc[2] + acc[3])

    def update_and_next(m_ref, krows, gd, nrows):
        acc = [None] * 4
        for j in range(HALF):
            mj = m_ref[j] + krows[j] * gd
            m_ref[j] = mj
            p = mj * nrows[j]
            acc[j % 4] = p if j < 4 else acc[j % 4] + p
        return (acc[0] + acc[1]) + (acc[2] + acc[3])

    def update(m_ref, krows, gd):
        for j in range(HALF):
            m_ref[j] = m_ref[j] + krows[j] * gd

    def deltas(k, q, vps, vpe, g, r):
        ns = jnp.sum(k * k, axis=0, keepdims=True) + 1e-6
        dvs = k - vps / ns
        ne = jnp.sum(q * q, axis=0, keepdims=True) + 1e-6
        dve = q - vpe / ne
        return g * dvs, ((1.0 - g) * r) * dve

    k0 = ks_ref[0]
    q0 = ke_ref[0]
    vps0 = matvec(ms_ref, rows(k0))
    vpe0 = matvec(me_ref, rows(q0))

    def body(s, carry):
        vps, vpe = carry
        k = ks_ref[s]
        q = ke_ref[s]
        g = g_ref[s]
        r = (t0 + s.astype(jnp.float32) + 1.0) * (1.0 / L)
        gd, ge = deltas(k, q, vps, vpe, g, r)
        nk = rows(ks_ref[s + 1])
        nq = rows(ke_ref[s + 1])
        vps = update_and_next(ms_ref, rows(k), gd, nk)
        vpe = update_and_next(me_ref, rows(q), ge, nq)
        return vps, vpe

    vps, vpe = jax.lax.fori_loop(0, n_steps - 1, body, (vps0, vpe0),
                                 unroll=False)

    sl = n_steps - 1
    kl = ks_ref[sl]
    ql = ke_ref[sl]
    gl = g_ref[sl]
    rl = (t0 + sl.astype(jnp.float32) + 1.0) * (1.0 / L)
    gd, ge = deltas(kl, ql, vps, vpe, gl, rl)
    update(ms_ref, rows(kl), gd)
    update(me_ref, rows(ql), ge)

    @pl.when(t == NTC - 1)
    def _():
        qs = ks_ref[TC - 1]
        qe = ke_ref[TC - 1]
        cs = matvec(ms_ref, rows(qs))
        ce = matvec(me_ref, rows(qe))
        cat = jnp.concatenate([cs, ce], axis=0)
        outT = jnp.dot(outwT_ref[...], cat, preferred_element_type=jnp.float32)
        out_ref[...] = outT.T + outb_ref[...]


def kernel(seq, embed, ff_w1, ff_b1, ff_w2, ff_b2, ln_g, ln_b,
           sem_w, sem_b, epi_w, epi_b, rtr_w, rtr_b, out_w, out_b):
    f32 = jnp.float32
    seq3 = jnp.transpose(seq).reshape(L // TSUB, 1, ENC_N)
    col = lambda v: v.reshape(-1, 1).astype(f32)

    n_enc = L // TSUB
    wspec = lambda shape: pl.BlockSpec(shape, lambda c, i: (0, 0))
    ksT, keT, gT = pl.pallas_call(
        _encoder_body,
        grid=(2, n_enc // 2),
        in_specs=[
            pl.BlockSpec((1, 1, ENC_N), lambda c, i: (c * (n_enc // 2) + i, 0, 0)),
            wspec((H, V)), wspec((2 * H, H)), wspec((2 * H, 1)),
            wspec((H, 2 * H)), wspec((H, 1)), wspec((H, 1)), wspec((H, 1)),
            wspec((HALF, H)), wspec((HALF, 1)), wspec((HALF, H)), wspec((HALF, 1)),
            wspec((1, H)), wspec((1, 1)),
        ],
        out_specs=[
            pl.BlockSpec((TSUB, HALF, B), lambda c, i: (c * (n_enc // 2) + i, 0, 0)),
            pl.BlockSpec((TSUB, HALF, B), lambda c, i: (c * (n_enc // 2) + i, 0, 0)),
            pl.BlockSpec((TSUB, 1, B), lambda c, i: (c * (n_enc // 2) + i, 0, 0)),
        ],
        out_shape=[
            jax.ShapeDtypeStruct((L, HALF, B), f32),
            jax.ShapeDtypeStruct((L, HALF, B), f32),
            jax.ShapeDtypeStruct((L, 1, B), f32),
        ],
        compiler_params=pltpu.CompilerParams(
            dimension_semantics=("parallel", "arbitrary"),
        ),
        name="router_encoder",
    )(seq3, jnp.transpose(embed), jnp.transpose(ff_w1), col(ff_b1),
      jnp.transpose(ff_w2), col(ff_b2), col(ln_g), col(ln_b),
      jnp.transpose(sem_w), col(sem_b), jnp.transpose(epi_w), col(epi_b),
      jnp.transpose(rtr_w), rtr_b.reshape(1, 1))

    BB = B // 2
    cspec = lambda shape: pl.BlockSpec(shape, lambda c, t: (0, 0))
    out = pl.pallas_call(
        _scan_body,
        grid=(2, NTC),
        in_specs=[
            pl.BlockSpec((TC, HALF, BB), lambda c, t: (t, 0, c)),
            pl.BlockSpec((TC, HALF, BB), lambda c, t: (t, 0, c)),
            pl.BlockSpec((TC, 1, BB), lambda c, t: (t, 0, c)),
            cspec((V, 2 * HALF)), cspec((1, V)),
        ],
        out_specs=pl.BlockSpec((BB, V), lambda c, t: (c, 0)),
        out_shape=jax.ShapeDtypeStruct((B, V), f32),
        scratch_shapes=[
            pltpu.VMEM((HALF, HALF, BB), f32),
            pltpu.VMEM((HALF, HALF, BB), f32),
        ],
        compiler_params=pltpu.CompilerParams(
            dimension_semantics=("parallel", "arbitrary"),
        ),
        name="router_scan",
    )(ksT, keT, gT, jnp.transpose(out_w), out_b.reshape(1, V))
    return out


# speedup vs baseline: 33.0205x; 1.0009x over previous
import functools

import jax
import jax.numpy as jnp
from jax.experimental import pallas as pl
from jax.experimental.pallas import tpu as pltpu

H = 64
V = 64
HALF = 32
B, L = 256, 2048

TSUB = 8
ENC_N = TSUB * B
TC = 128
NTC = L // TC


def _encoder_body(seq_ref, embT_ref, w1T_ref, b1_ref, w2T_ref, b2_ref,
                  lng_ref, lnb_ref, semT_ref, semb_ref, epiT_ref, epib_ref,
                  rtrT_ref, rtrb_ref, ks_ref, ke_ref, g_ref):
    seq_row = seq_ref[0]
    iota_v = jax.lax.broadcasted_iota(jnp.int32, (V, ENC_N), 0)
    onehot = jnp.where(iota_v == seq_row, 1.0, 0.0)
    e = jnp.dot(embT_ref[...], onehot, preferred_element_type=jnp.float32)

    a1 = jnp.maximum(
        jnp.dot(w1T_ref[...], e, preferred_element_type=jnp.float32)
        + b1_ref[...], 0.0)
    f = jnp.dot(w2T_ref[...], a1, preferred_element_type=jnp.float32) + b2_ref[...]
    x = e + f
    mu = jnp.mean(x, axis=0, keepdims=True)
    xc = x - mu
    var = jnp.mean(xc * xc, axis=0, keepdims=True)
    h = xc * jax.lax.rsqrt(var + 1e-5) * lng_ref[...] + lnb_ref[...]

    ks = jnp.dot(semT_ref[...], h, preferred_element_type=jnp.float32) + semb_ref[...]
    ke = jnp.dot(epiT_ref[...], h, preferred_element_type=jnp.float32) + epib_ref[...]
    gr = jax.nn.sigmoid(
        jnp.dot(rtrT_ref[...], h, preferred_element_type=jnp.float32) + rtrb_ref[...])

    for s in range(TSUB):
        sl = slice(s * B, (s + 1) * B)
        ks_ref[s] = ks[:, sl]
        ke_ref[s] = ke[:, sl]
        g_ref[s] = gr[:, sl]


def _scan_body(ks_ref, ke_ref, g_ref, outwT_ref, outb_ref, out_ref,
               ms_ref, me_ref):
    t = pl.program_id(1)

    @pl.when(t == 0)
    def _():
        ms_ref[...] = jnp.zeros_like(ms_ref)
        me_ref[...] = jnp.zeros_like(me_ref)

    n_steps = jnp.where(t == NTC - 1, TC - 1, TC)
    t0 = (t * TC).astype(jnp.float32)

    def rows(k):
        return [k[j:j + 1, :] for j in range(HALF)]

    def matvec(m_ref, krows):
        acc = [m_ref[j] * krows[j] for j in range(4)]
        for j in range(4, HALF):
            acc[j % 4] = acc[j % 4] + m_ref[j] * krows[j]
        return (acc[0] + acc[1]) + (acc[2] + acc[3])

    def update_and_next(m_ref, krows, gd, nrows):
        acc = [None] * 4
        for j in range(HALF):
            mj = m_ref[j] + krows[j] * gd
            m_ref[j] = mj
            p = mj * nrows[j]
            acc[j % 4] = p if j < 4 else acc[j % 4] + p
        return (acc[0] + acc[1]) + (acc[2] + acc[3])

    def update(m_ref, krows, gd):
        for j in range(HALF):
            m_ref[j] = m_ref[j] + krows[j] * gd

    def deltas(k, q, vps, vpe, g, r):
        ns = jnp.sum(k * k, axis=0, keepdims=True) + 1e-6
        dvs = k - vps / ns
        ne = jnp.sum(q * q, axis=0, keepdims=True) + 1e-6
        dve = q - vpe / ne
        return g * dvs, ((1.0 - g) * r) * dve

    k0 = ks_ref[0]
    q0 = ke_ref[0]
    vps0 = matvec(ms_ref, rows(k0))
    vpe0 = matvec(me_ref, rows(q0))

    def body(s, carry):
        vps, vpe = carry
        k = ks_ref[s]
        q = ke_ref[s]
        g = g_ref[s]
        r = (t0 + s.astype(jnp.float32) + 1.0) * (1.0 / L)
        gd, ge = deltas(k, q, vps, vpe, g, r)
        nk = rows(ks_ref[s + 1])
        nq = rows(ke_ref[s + 1])
        vps = update_and_next(ms_ref, rows(k), gd, nk)
        vpe = update_and_next(me_ref, rows(q), ge, nq)
        return vps, vpe

    vps, vpe = jax.lax.fori_loop(0, n_steps - 1, body, (vps0, vpe0),
                                 unroll=False)

    sl = n_steps - 1
    kl = ks_ref[sl]
    ql = ke_ref[sl]
    gl = g_ref[sl]
    rl = (t0 + sl.astype(jnp.float32) + 1.0) * (1.0 / L)
    gd, ge = deltas(kl, ql, vps, vpe, gl, rl)
    update(ms_ref, rows(kl), gd)
    update(me_ref, rows(ql), ge)

    @pl.when(t == NTC - 1)
    def _():
        qs = ks_ref[TC - 1]
        qe = ke_ref[TC - 1]
        cs = matvec(ms_ref, rows(qs))
        ce = matvec(me_ref, rows(qe))
        cat = jnp.concatenate([cs, ce], axis=0)
        outT = jnp.dot(outwT_ref[...], cat, preferred_element_type=jnp.float32)
        out_ref[...] = outT.T + outb_ref[...]


def kernel(seq, embed, ff_w1, ff_b1, ff_w2, ff_b2, ln_g, ln_b,
           sem_w, sem_b, epi_w, epi_b, rtr_w, rtr_b, out_w, out_b):
    f32 = jnp.float32
    seq3 = jnp.transpose(seq).reshape(L // TSUB, 1, ENC_N)
    col = lambda v: v.reshape(-1, 1).astype(f32)

    n_enc = L // TSUB
    wspec = lambda shape: pl.BlockSpec(shape, lambda c, i: (0, 0))
    ksT, keT, gT = pl.pallas_call(
        _encoder_body,
        grid=(2, n_enc // 2),
        in_specs=[
            pl.BlockSpec((1, 1, ENC_N), lambda c, i: (c * (n_enc // 2) + i, 0, 0)),
            wspec((H, V)), wspec((2 * H, H)), wspec((2 * H, 1)),
            wspec((H, 2 * H)), wspec((H, 1)), wspec((H, 1)), wspec((H, 1)),
            wspec((HALF, H)), wspec((HALF, 1)), wspec((HALF, H)), wspec((HALF, 1)),
            wspec((1, H)), wspec((1, 1)),
        ],
        out_specs=[
            pl.BlockSpec((TSUB, HALF, B), lambda c, i: (c * (n_enc // 2) + i, 0, 0)),
            pl.BlockSpec((TSUB, HALF, B), lambda c, i: (c * (n_enc // 2) + i, 0, 0)),
            pl.BlockSpec((TSUB, 1, B), lambda c, i: (c * (n_enc // 2) + i, 0, 0)),
        ],
        out_shape=[
            jax.ShapeDtypeStruct((L, HALF, B), f32),
            jax.ShapeDtypeStruct((L, HALF, B), f32),
            jax.ShapeDtypeStruct((L, 1, B), f32),
        ],
        compiler_params=pltpu.CompilerParams(
            dimension_semantics=("parallel", "arbitrary"),
        ),
        name="router_encoder",
    )(seq3, jnp.transpose(embed), jnp.transpose(ff_w1), col(ff_b1),
      jnp.transpose(ff_w2), col(ff_b2), col(ln_g), col(ln_b),
      jnp.transpose(sem_w), col(sem_b), jnp.transpose(epi_w), col(epi_b),
      jnp.transpose(rtr_w), rtr_b.reshape(1, 1))

    BB = B // 2
    cspec = lambda shape: pl.BlockSpec(shape, lambda c, t: (0, 0))
    out = pl.pallas_call(
        _scan_body,
        grid=(2, NTC),
        in_specs=[
            pl.BlockSpec((TC, HALF, BB), lambda c, t: (t, 0, c)),
            pl.BlockSpec((TC, HALF, BB), lambda c, t: (t, 0, c)),
            pl.BlockSpec((TC, 1, BB), lambda c, t: (t, 0, c)),
            cspec((V, 2 * HALF)), cspec((1, V)),
        ],
        out_specs=pl.BlockSpec((BB, V), lambda c, t: (c, 0)),
        out_shape=jax.ShapeDtypeStruct((B, V), f32),
        scratch_shapes=[
            pltpu.VMEM((HALF, HALF, BB), f32),
            pltpu.VMEM((HALF, HALF, BB), f32),
        ],
        compiler_params=pltpu.CompilerParams(
            dimension_semantics=("parallel", "arbitrary"),
            flags={"XLA_TPU_STORE_TO_LOAD_FORWARDING_WINDOW": 8192},
        ),
        name="router_scan",
    )(ksT, keT, gT, jnp.transpose(out_w), out_b.reshape(1, V))
    return out


# speedup vs baseline: 39.1571x; 1.1858x over previous
import functools

import jax
import jax.numpy as jnp
from jax.experimental import pallas as pl
from jax.experimental.pallas import tpu as pltpu

H = 64
V = 64
HALF = 32
B, L = 256, 2048

TSUB = 8
ENC_N = TSUB * B
TC = 128
NTC = L // TC
P = 4
NB = TC // P


def _encoder_body(seq_ref, embT_ref, w1T_ref, b1_ref, w2T_ref, b2_ref,
                  lng_ref, lnb_ref, semT_ref, semb_ref, epiT_ref, epib_ref,
                  rtrT_ref, rtrb_ref, ks_ref, ke_ref, g_ref):
    seq_row = seq_ref[0]
    iota_v = jax.lax.broadcasted_iota(jnp.int32, (V, ENC_N), 0)
    onehot = jnp.where(iota_v == seq_row, 1.0, 0.0)
    e = jnp.dot(embT_ref[...], onehot, preferred_element_type=jnp.float32)

    a1 = jnp.maximum(
        jnp.dot(w1T_ref[...], e, preferred_element_type=jnp.float32)
        + b1_ref[...], 0.0)
    f = jnp.dot(w2T_ref[...], a1, preferred_element_type=jnp.float32) + b2_ref[...]
    x = e + f
    mu = jnp.mean(x, axis=0, keepdims=True)
    xc = x - mu
    var = jnp.mean(xc * xc, axis=0, keepdims=True)
    h = xc * jax.lax.rsqrt(var + 1e-5) * lng_ref[...] + lnb_ref[...]

    ks = jnp.dot(semT_ref[...], h, preferred_element_type=jnp.float32) + semb_ref[...]
    ke = jnp.dot(epiT_ref[...], h, preferred_element_type=jnp.float32) + epib_ref[...]
    gr = jax.nn.sigmoid(
        jnp.dot(rtrT_ref[...], h, preferred_element_type=jnp.float32) + rtrb_ref[...])

    for s in range(TSUB):
        sl = slice(s * B, (s + 1) * B)
        ks_ref[s] = ks[:, sl]
        ke_ref[s] = ke[:, sl]
        g_ref[s] = gr[:, sl]


def _scan_body(ks_ref, ke_ref, g_ref, outwT_ref, outb_ref, out_ref,
               ms_ref, me_ref):
    t = pl.program_id(1)

    @pl.when(t == 0)
    def _():
        ms_ref[...] = jnp.zeros_like(ms_ref)
        me_ref[...] = jnp.zeros_like(me_ref)

    t0 = (t * TC).astype(jnp.float32)

    def sumj(x):
        return jnp.sum(x, axis=0, keepdims=True)

    def rows(k):
        return [k[j:j + 1, :] for j in range(HALF)]

    def load_block(ref, b):
        return [ref[b * P + i] for i in range(P)]

    def w0_pass(m_ref, rws):
        accs = [None] * len(rws)
        for j in range(HALF):
            mj = m_ref[j]
            for tt in range(len(rws)):
                p = mj * rws[tt][j]
                accs[tt] = p if j == 0 else accs[tt] + p
        return accs

    def gates(b, n):
        a_s, a_e = [], []
        for i in range(P):
            if i >= n:
                break
            gg = g_ref[b * P + i]
            rr = (t0 + (b * P + i + 1) * 1.0) * (1.0 / L)
            a_s.append(gg)
            a_e.append((1.0 - gg) * rr)
        return a_s, a_e

    def make_us(kb, w0, a_l, n):
        nn = [sumj(kb[i] * kb[i]) + 1e-6 for i in range(n)]
        d = {}
        for ss in range(n):
            for tt in range(ss + 1, n):
                d[(ss, tt)] = sumj(kb[ss] * kb[tt])
        us = []
        for tt in range(n):
            w = w0[tt]
            for ss in range(tt):
                w = w + d[(ss, tt)] * us[ss]
            ci = a_l[tt] / nn[tt]
            us.append(a_l[tt] * kb[tt] - ci * w)
        return us

    def upd_pass(m_ref, rws, us, nrws):
        accs = [None] * P if nrws is not None else None
        for j in range(HALF):
            mj = m_ref[j]
            for tt in range(len(us)):
                mj = mj + us[tt] * rws[tt][j]
            m_ref[j] = mj
            if nrws is not None:
                for tt in range(P):
                    p = mj * nrws[tt][j]
                    accs[tt] = p if j == 0 else accs[tt] + p
        return accs

    kb0 = load_block(ks_ref, 0)
    qb0 = load_block(ke_ref, 0)
    w0s = w0_pass(ms_ref, [rows(k) for k in kb0])
    w0e = w0_pass(me_ref, [rows(q) for q in qb0])

    def body(b, carry):
        w0s, w0e = list(carry[:P]), list(carry[P:])
        a_s, a_e = gates(b, P)
        kb = load_block(ks_ref, b)
        us = make_us(kb, w0s, a_s, P)
        nrws = [rows(k) for k in load_block(ks_ref, b + 1)]
        nw0s = upd_pass(ms_ref, [rows(k) for k in kb], us, nrws)
        qb = load_block(ke_ref, b)
        ue = make_us(qb, w0e, a_e, P)
        nrwe = [rows(q) for q in load_block(ke_ref, b + 1)]
        nw0e = upd_pass(me_ref, [rows(q) for q in qb], ue, nrwe)
        return tuple(nw0s) + tuple(nw0e)

    carry = jax.lax.fori_loop(0, NB - 1, body, tuple(w0s) + tuple(w0e),
                              unroll=False)
    w0s, w0e = list(carry[:P]), list(carry[P:])

    bl = NB - 1
    kb = load_block(ks_ref, bl)
    qb = load_block(ke_ref, bl)

    @pl.when(t < NTC - 1)
    def _():
        a_s, a_e = gates(bl, P)
        us = make_us(kb, w0s, a_s, P)
        upd_pass(ms_ref, [rows(k) for k in kb], us, None)
        ue = make_us(qb, w0e, a_e, P)
        upd_pass(me_ref, [rows(q) for q in qb], ue, None)

    @pl.when(t == NTC - 1)
    def _():
        a_s, a_e = gates(bl, P - 1)
        us = make_us(kb, w0s, a_s, P - 1)
        upd_pass(ms_ref, [rows(k) for k in kb[:P - 1]], us, None)
        ue = make_us(qb, w0e, a_e, P - 1)
        upd_pass(me_ref, [rows(q) for q in qb[:P - 1]], ue, None)

        cs = w0_pass(ms_ref, [rows(kb[P - 1])])[0]
        ce = w0_pass(me_ref, [rows(qb[P - 1])])[0]
        cat = jnp.concatenate([cs, ce], axis=0)
        outT = jnp.dot(outwT_ref[...], cat, preferred_element_type=jnp.float32)
        out_ref[...] = outT.T + outb_ref[...]


def kernel(seq, embed, ff_w1, ff_b1, ff_w2, ff_b2, ln_g, ln_b,
           sem_w, sem_b, epi_w, epi_b, rtr_w, rtr_b, out_w, out_b):
    f32 = jnp.float32
    seq3 = jnp.transpose(seq).reshape(L // TSUB, 1, ENC_N)
    col = lambda v: v.reshape(-1, 1).astype(f32)

    n_enc = L // TSUB
    wspec = lambda shape: pl.BlockSpec(shape, lambda c, i: (0, 0))
    ksT, keT, gT = pl.pallas_call(
        _encoder_body,
        grid=(2, n_enc // 2),
        in_specs=[
            pl.BlockSpec((1, 1, ENC_N), lambda c, i: (c * (n_enc // 2) + i, 0, 0)),
            wspec((H, V)), wspec((2 * H, H)), wspec((2 * H, 1)),
            wspec((H, 2 * H)), wspec((H, 1)), wspec((H, 1)), wspec((H, 1)),
            wspec((HALF, H)), wspec((HALF, 1)), wspec((HALF, H)), wspec((HALF, 1)),
            wspec((1, H)), wspec((1, 1)),
        ],
        out_specs=[
            pl.BlockSpec((TSUB, HALF, B), lambda c, i: (c * (n_enc // 2) + i, 0, 0)),
            pl.BlockSpec((TSUB, HALF, B), lambda c, i: (c * (n_enc // 2) + i, 0, 0)),
            pl.BlockSpec((TSUB, 1, B), lambda c, i: (c * (n_enc // 2) + i, 0, 0)),
        ],
        out_shape=[
            jax.ShapeDtypeStruct((L, HALF, B), f32),
            jax.ShapeDtypeStruct((L, HALF, B), f32),
            jax.ShapeDtypeStruct((L, 1, B), f32),
        ],
        compiler_params=pltpu.CompilerParams(
            dimension_semantics=("parallel", "arbitrary"),
        ),
        name="router_encoder",
    )(seq3, jnp.transpose(embed), jnp.transpose(ff_w1), col(ff_b1),
      jnp.transpose(ff_w2), col(ff_b2), col(ln_g), col(ln_b),
      jnp.transpose(sem_w), col(sem_b), jnp.transpose(epi_w), col(epi_b),
      jnp.transpose(rtr_w), rtr_b.reshape(1, 1))

    BB = B // 2
    cspec = lambda shape: pl.BlockSpec(shape, lambda c, t: (0, 0))
    out = pl.pallas_call(
        _scan_body,
        grid=(2, NTC),
        in_specs=[
            pl.BlockSpec((TC, HALF, BB), lambda c, t: (t, 0, c)),
            pl.BlockSpec((TC, HALF, BB), lambda c, t: (t, 0, c)),
            pl.BlockSpec((TC, 1, BB), lambda c, t: (t, 0, c)),
            cspec((V, 2 * HALF)), cspec((1, V)),
        ],
        out_specs=pl.BlockSpec((BB, V), lambda c, t: (c, 0)),
        out_shape=jax.ShapeDtypeStruct((B, V), f32),
        scratch_shapes=[
            pltpu.VMEM((HALF, HALF, BB), f32),
            pltpu.VMEM((HALF, HALF, BB), f32),
        ],
        compiler_params=pltpu.CompilerParams(
            dimension_semantics=("parallel", "arbitrary"),
        ),
        name="router_scan",
    )(ksT, keT, gT, jnp.transpose(out_w), out_b.reshape(1, V))
    return out


# speedup vs baseline: 39.1826x; 1.0007x over previous
import functools

import jax
import jax.numpy as jnp
from jax.experimental import pallas as pl
from jax.experimental.pallas import tpu as pltpu

H = 64
V = 64
HALF = 32
B, L = 256, 2048

TSUB = 8
ENC_N = TSUB * B
TC = 128
NTC = L // TC
P = 4
NB = TC // P


def _encoder_body(seq_ref, embT_ref, w1T_ref, b1_ref, w2T_ref, b2_ref,
                  lng_ref, lnb_ref, semT_ref, semb_ref, epiT_ref, epib_ref,
                  rtrT_ref, rtrb_ref, ks_ref, ke_ref, g_ref):
    seq_row = seq_ref[0]
    iota_v = jax.lax.broadcasted_iota(jnp.int32, (V, ENC_N), 0)
    onehot = jnp.where(iota_v == seq_row, 1.0, 0.0)
    e = jnp.dot(embT_ref[...], onehot, preferred_element_type=jnp.float32)

    a1 = jnp.maximum(
        jnp.dot(w1T_ref[...], e, preferred_element_type=jnp.float32)
        + b1_ref[...], 0.0)
    f = jnp.dot(w2T_ref[...], a1, preferred_element_type=jnp.float32) + b2_ref[...]
    x = e + f
    mu = jnp.mean(x, axis=0, keepdims=True)
    xc = x - mu
    var = jnp.mean(xc * xc, axis=0, keepdims=True)
    h = xc * jax.lax.rsqrt(var + 1e-5) * lng_ref[...] + lnb_ref[...]

    ks = jnp.dot(semT_ref[...], h, preferred_element_type=jnp.float32) + semb_ref[...]
    ke = jnp.dot(epiT_ref[...], h, preferred_element_type=jnp.float32) + epib_ref[...]
    gr = jax.nn.sigmoid(
        jnp.dot(rtrT_ref[...], h, preferred_element_type=jnp.float32) + rtrb_ref[...])

    for s in range(TSUB):
        sl = slice(s * B, (s + 1) * B)
        ks_ref[s] = ks[:, sl]
        ke_ref[s] = ke[:, sl]
        g_ref[s] = gr[:, sl]


def _scan_body(ks_ref, ke_ref, g_ref, outwT_ref, outb_ref, out_ref,
               ms_ref, me_ref):
    t = pl.program_id(1)

    @pl.when(t == 0)
    def _():
        ms_ref[...] = jnp.zeros_like(ms_ref)
        me_ref[...] = jnp.zeros_like(me_ref)

    t0 = (t * TC).astype(jnp.float32)

    def sumj(x):
        return jnp.sum(x, axis=0, keepdims=True)

    def rows(k):
        return [k[j:j + 1, :] for j in range(HALF)]

    def load_block(ref, b):
        return [ref[b * P + i] for i in range(P)]

    def row_lists(ref, b, n=P):
        return [[ref[b * P + i, j:j + 1, :] for j in range(HALF)]
                for i in range(n)]

    def w0_pass(m_ref, rws):
        accs = [None] * len(rws)
        for j in range(HALF):
            mj = m_ref[j]
            for tt in range(len(rws)):
                p = mj * rws[tt][j]
                accs[tt] = p if j == 0 else accs[tt] + p
        return accs

    def gates(b, n):
        a_s, a_e = [], []
        for i in range(P):
            if i >= n:
                break
            gg = g_ref[b * P + i]
            rr = (t0 + (b * P + i + 1) * 1.0) * (1.0 / L)
            a_s.append(gg)
            a_e.append((1.0 - gg) * rr)
        return a_s, a_e

    def make_us(kb, w0, a_l, n):
        nn = [sumj(kb[i] * kb[i]) + 1e-6 for i in range(n)]
        d = {}
        for ss in range(n):
            for tt in range(ss + 1, n):
                d[(ss, tt)] = sumj(kb[ss] * kb[tt])
        us = []
        for tt in range(n):
            w = w0[tt]
            for ss in range(tt):
                w = w + d[(ss, tt)] * us[ss]
            ci = a_l[tt] / nn[tt]
            us.append(a_l[tt] * kb[tt] - ci * w)
        return us

    def upd_pass(m_ref, rws, us, nrws):
        accs = [None] * P if nrws is not None else None
        for j in range(HALF):
            mj = m_ref[j]
            for tt in range(len(us)):
                mj = mj + us[tt] * rws[tt][j]
            m_ref[j] = mj
            if nrws is not None:
                for tt in range(P):
                    p = mj * nrws[tt][j]
                    accs[tt] = p if j == 0 else accs[tt] + p
        return accs

    w0s = w0_pass(ms_ref, row_lists(ks_ref, 0))
    w0e = w0_pass(me_ref, row_lists(ke_ref, 0))

    def body(b, carry):
        w0s, w0e = list(carry[:P]), list(carry[P:])
        a_s, a_e = gates(b, P)
        kb = load_block(ks_ref, b)
        us = make_us(kb, w0s, a_s, P)
        nw0s = upd_pass(ms_ref, row_lists(ks_ref, b), us,
                        row_lists(ks_ref, b + 1))
        qb = load_block(ke_ref, b)
        ue = make_us(qb, w0e, a_e, P)
        nw0e = upd_pass(me_ref, row_lists(ke_ref, b), ue,
                        row_lists(ke_ref, b + 1))
        return tuple(nw0s) + tuple(nw0e)

    carry = jax.lax.fori_loop(0, NB - 1, body, tuple(w0s) + tuple(w0e),
                              unroll=False)
    w0s, w0e = list(carry[:P]), list(carry[P:])

    bl = NB - 1
    kb = load_block(ks_ref, bl)
    qb = load_block(ke_ref, bl)

    @pl.when(t < NTC - 1)
    def _():
        a_s, a_e = gates(bl, P)
        us = make_us(kb, w0s, a_s, P)
        upd_pass(ms_ref, row_lists(ks_ref, bl), us, None)
        ue = make_us(qb, w0e, a_e, P)
        upd_pass(me_ref, row_lists(ke_ref, bl), ue, None)

    @pl.when(t == NTC - 1)
    def _():
        a_s, a_e = gates(bl, P - 1)
        us = make_us(kb, w0s, a_s, P - 1)
        upd_pass(ms_ref, row_lists(ks_ref, bl, P - 1), us, None)
        ue = make_us(qb, w0e, a_e, P - 1)
        upd_pass(me_ref, row_lists(ke_ref, bl, P - 1), ue, None)

        cs = w0_pass(ms_ref, [rows(kb[P - 1])])[0]
        ce = w0_pass(me_ref, [rows(qb[P - 1])])[0]
        cat = jnp.concatenate([cs, ce], axis=0)
        outT = jnp.dot(outwT_ref[...], cat, preferred_element_type=jnp.float32)
        out_ref[...] = outT.T + outb_ref[...]


def kernel(seq, embed, ff_w1, ff_b1, ff_w2, ff_b2, ln_g, ln_b,
           sem_w, sem_b, epi_w, epi_b, rtr_w, rtr_b, out_w, out_b):
    f32 = jnp.float32
    seq3 = jnp.transpose(seq).reshape(L // TSUB, 1, ENC_N)
    col = lambda v: v.reshape(-1, 1).astype(f32)

    n_enc = L // TSUB
    wspec = lambda shape: pl.BlockSpec(shape, lambda c, i: (0, 0))
    ksT, keT, gT = pl.pallas_call(
        _encoder_body,
        grid=(2, n_enc // 2),
        in_specs=[
            pl.BlockSpec((1, 1, ENC_N), lambda c, i: (c * (n_enc // 2) + i, 0, 0)),
            wspec((H, V)), wspec((2 * H, H)), wspec((2 * H, 1)),
            wspec((H, 2 * H)), wspec((H, 1)), wspec((H, 1)), wspec((H, 1)),
            wspec((HALF, H)), wspec((HALF, 1)), wspec((HALF, H)), wspec((HALF, 1)),
            wspec((1, H)), wspec((1, 1)),
        ],
        out_specs=[
            pl.BlockSpec((TSUB, HALF, B), lambda c, i: (c * (n_enc // 2) + i, 0, 0)),
            pl.BlockSpec((TSUB, HALF, B), lambda c, i: (c * (n_enc // 2) + i, 0, 0)),
            pl.BlockSpec((TSUB, 1, B), lambda c, i: (c * (n_enc // 2) + i, 0, 0)),
        ],
        out_shape=[
            jax.ShapeDtypeStruct((L, HALF, B), f32),
            jax.ShapeDtypeStruct((L, HALF, B), f32),
            jax.ShapeDtypeStruct((L, 1, B), f32),
        ],
        compiler_params=pltpu.CompilerParams(
            dimension_semantics=("parallel", "arbitrary"),
        ),
        name="router_encoder",
    )(seq3, jnp.transpose(embed), jnp.transpose(ff_w1), col(ff_b1),
      jnp.transpose(ff_w2), col(ff_b2), col(ln_g), col(ln_b),
      jnp.transpose(sem_w), col(sem_b), jnp.transpose(epi_w), col(epi_b),
      jnp.transpose(rtr_w), rtr_b.reshape(1, 1))

    BB = B // 2
    cspec = lambda shape: pl.BlockSpec(shape, lambda c, t: (0, 0))
    out = pl.pallas_call(
        _scan_body,
        grid=(2, NTC),
        in_specs=[
            pl.BlockSpec((TC, HALF, BB), lambda c, t: (t, 0, c)),
            pl.BlockSpec((TC, HALF, BB), lambda c, t: (t, 0, c)),
            pl.BlockSpec((TC, 1, BB), lambda c, t: (t, 0, c)),
            cspec((V, 2 * HALF)), cspec((1, V)),
        ],
        out_specs=pl.BlockSpec((BB, V), lambda c, t: (c, 0)),
        out_shape=jax.ShapeDtypeStruct((B, V), f32),
        scratch_shapes=[
            pltpu.VMEM((HALF, HALF, BB), f32),
            pltpu.VMEM((HALF, HALF, BB), f32),
        ],
        compiler_params=pltpu.CompilerParams(
            dimension_semantics=("parallel", "arbitrary"),
        ),
        name="router_scan",
    )(ksT, keT, gT, jnp.transpose(out_w), out_b.reshape(1, V))
    return out


# speedup vs baseline: 43.1379x; 1.1009x over previous
import functools

import jax
import jax.numpy as jnp
from jax.experimental import pallas as pl
from jax.experimental.pallas import tpu as pltpu

H = 64
V = 64
HALF = 32
B, L = 256, 2048

TSUB = 16
ENC_N = TSUB * B
TC = 128
NTC = L // TC
P = 4
NB = TC // P


def _encoder_body(seq_ref, embT_ref, w1T_ref, b1_ref, w2T_ref, b2_ref,
                  lng_ref, lnb_ref, semT_ref, semb_ref, epiT_ref, epib_ref,
                  rtrT_ref, rtrb_ref, ks_ref, ke_ref, g_ref):
    seq_row = seq_ref[0]
    iota_v = jax.lax.broadcasted_iota(jnp.int32, (V, ENC_N), 0)
    onehot = jnp.where(iota_v == seq_row, 1.0, 0.0)
    e = jnp.dot(embT_ref[...], onehot, preferred_element_type=jnp.float32)

    a1 = jnp.maximum(
        jnp.dot(w1T_ref[...], e, preferred_element_type=jnp.float32)
        + b1_ref[...], 0.0)
    f = jnp.dot(w2T_ref[...], a1, preferred_element_type=jnp.float32) + b2_ref[...]
    x = e + f
    mu = jnp.mean(x, axis=0, keepdims=True)
    xc = x - mu
    var = jnp.mean(xc * xc, axis=0, keepdims=True)
    h = xc * jax.lax.rsqrt(var + 1e-5) * lng_ref[...] + lnb_ref[...]

    ks = jnp.dot(semT_ref[...], h, preferred_element_type=jnp.float32) + semb_ref[...]
    ke = jnp.dot(epiT_ref[...], h, preferred_element_type=jnp.float32) + epib_ref[...]
    gr = jax.nn.sigmoid(
        jnp.dot(rtrT_ref[...], h, preferred_element_type=jnp.float32) + rtrb_ref[...])

    for s in range(TSUB):
        sl = slice(s * B, (s + 1) * B)
        ks_ref[s] = ks[:, sl]
        ke_ref[s] = ke[:, sl]
        g_ref[s] = gr[:, sl]


def _scan_body(ks_ref, ke_ref, g_ref, outwT_ref, outb_ref, out_ref,
               ms_ref, me_ref):
    t = pl.program_id(1)

    @pl.when(t == 0)
    def _():
        ms_ref[...] = jnp.zeros_like(ms_ref)
        me_ref[...] = jnp.zeros_like(me_ref)

    t0 = (t * TC).astype(jnp.float32)

    def sumj(x):
        return jnp.sum(x, axis=0, keepdims=True)

    def rows(k):
        return [k[j:j + 1, :] for j in range(HALF)]

    def load_block(ref, b):
        return [ref[b * P + i] for i in range(P)]

    def row_lists(ref, b, n=P):
        return [[ref[b * P + i, j:j + 1, :] for j in range(HALF)]
                for i in range(n)]

    def w0_pass(m_ref, rws):
        accs = [None] * len(rws)
        for j in range(HALF):
            mj = m_ref[j]
            for tt in range(len(rws)):
                p = mj * rws[tt][j]
                accs[tt] = p if j == 0 else accs[tt] + p
        return accs

    def gates(b, n):
        a_s, a_e = [], []
        for i in range(P):
            if i >= n:
                break
            gg = g_ref[b * P + i]
            rr = (t0 + (b * P + i + 1) * 1.0) * (1.0 / L)
            a_s.append(gg)
            a_e.append((1.0 - gg) * rr)
        return a_s, a_e

    def make_us(kb, w0, a_l, n):
        nn = [sumj(kb[i] * kb[i]) + 1e-6 for i in range(n)]
        d = {}
        for ss in range(n):
            for tt in range(ss + 1, n):
                d[(ss, tt)] = sumj(kb[ss] * kb[tt])
        us = []
        for tt in range(n):
            w = w0[tt]
            for ss in range(tt):
                w = w + d[(ss, tt)] * us[ss]
            ci = a_l[tt] / nn[tt]
            us.append(a_l[tt] * kb[tt] - ci * w)
        return us

    def upd_pass(m_ref, rws, us, nrws):
        accs = [None] * P if nrws is not None else None
        for j in range(HALF):
            mj = m_ref[j]
            for tt in range(len(us)):
                mj = mj + us[tt] * rws[tt][j]
            m_ref[j] = mj
            if nrws is not None:
                for tt in range(P):
                    p = mj * nrws[tt][j]
                    accs[tt] = p if j == 0 else accs[tt] + p
        return accs

    w0s = w0_pass(ms_ref, row_lists(ks_ref, 0))
    w0e = w0_pass(me_ref, row_lists(ke_ref, 0))

    def body(b, carry):
        w0s, w0e = list(carry[:P]), list(carry[P:])
        a_s, a_e = gates(b, P)
        kb = load_block(ks_ref, b)
        us = make_us(kb, w0s, a_s, P)
        nw0s = upd_pass(ms_ref, row_lists(ks_ref, b), us,
                        row_lists(ks_ref, b + 1))
        qb = load_block(ke_ref, b)
        ue = make_us(qb, w0e, a_e, P)
        nw0e = upd_pass(me_ref, row_lists(ke_ref, b), ue,
                        row_lists(ke_ref, b + 1))
        return tuple(nw0s) + tuple(nw0e)

    carry = jax.lax.fori_loop(0, NB - 1, body, tuple(w0s) + tuple(w0e),
                              unroll=False)
    w0s, w0e = list(carry[:P]), list(carry[P:])

    bl = NB - 1
    kb = load_block(ks_ref, bl)
    qb = load_block(ke_ref, bl)

    @pl.when(t < NTC - 1)
    def _():
        a_s, a_e = gates(bl, P)
        us = make_us(kb, w0s, a_s, P)
        upd_pass(ms_ref, row_lists(ks_ref, bl), us, None)
        ue = make_us(qb, w0e, a_e, P)
        upd_pass(me_ref, row_lists(ke_ref, bl), ue, None)

    @pl.when(t == NTC - 1)
    def _():
        a_s, a_e = gates(bl, P - 1)
        us = make_us(kb, w0s, a_s, P - 1)
        upd_pass(ms_ref, row_lists(ks_ref, bl, P - 1), us, None)
        ue = make_us(qb, w0e, a_e, P - 1)
        upd_pass(me_ref, row_lists(ke_ref, bl, P - 1), ue, None)

        cs = w0_pass(ms_ref, [rows(kb[P - 1])])[0]
        ce = w0_pass(me_ref, [rows(qb[P - 1])])[0]
        cat = jnp.concatenate([cs, ce], axis=0)
        outT = jnp.dot(outwT_ref[...], cat, preferred_element_type=jnp.float32)
        out_ref[...] = outT.T + outb_ref[...]


def kernel(seq, embed, ff_w1, ff_b1, ff_w2, ff_b2, ln_g, ln_b,
           sem_w, sem_b, epi_w, epi_b, rtr_w, rtr_b, out_w, out_b):
    f32 = jnp.float32
    seq3 = jnp.transpose(seq).reshape(L // TSUB, 1, ENC_N)
    col = lambda v: v.reshape(-1, 1).astype(f32)

    n_enc = L // TSUB
    wspec = lambda shape: pl.BlockSpec(shape, lambda c, i: (0, 0))
    ksT, keT, gT = pl.pallas_call(
        _encoder_body,
        grid=(2, n_enc // 2),
        in_specs=[
            pl.BlockSpec((1, 1, ENC_N), lambda c, i: (c * (n_enc // 2) + i, 0, 0)),
            wspec((H, V)), wspec((2 * H, H)), wspec((2 * H, 1)),
            wspec((H, 2 * H)), wspec((H, 1)), wspec((H, 1)), wspec((H, 1)),
            wspec((HALF, H)), wspec((HALF, 1)), wspec((HALF, H)), wspec((HALF, 1)),
            wspec((1, H)), wspec((1, 1)),
        ],
        out_specs=[
            pl.BlockSpec((TSUB, HALF, B), lambda c, i: (c * (n_enc // 2) + i, 0, 0)),
            pl.BlockSpec((TSUB, HALF, B), lambda c, i: (c * (n_enc // 2) + i, 0, 0)),
            pl.BlockSpec((TSUB, 1, B), lambda c, i: (c * (n_enc // 2) + i, 0, 0)),
        ],
        out_shape=[
            jax.ShapeDtypeStruct((L, HALF, B), f32),
            jax.ShapeDtypeStruct((L, HALF, B), f32),
            jax.ShapeDtypeStruct((L, 1, B), f32),
        ],
        compiler_params=pltpu.CompilerParams(
            dimension_semantics=("parallel", "arbitrary"),
        ),
        name="router_encoder",
    )(seq3, jnp.transpose(embed), jnp.transpose(ff_w1), col(ff_b1),
      jnp.transpose(ff_w2), col(ff_b2), col(ln_g), col(ln_b),
      jnp.transpose(sem_w), col(sem_b), jnp.transpose(epi_w), col(epi_b),
      jnp.transpose(rtr_w), rtr_b.reshape(1, 1))

    BB = B // 2
    cspec = lambda shape: pl.BlockSpec(shape, lambda c, t: (0, 0))
    out = pl.pallas_call(
        _scan_body,
        grid=(2, NTC),
        in_specs=[
            pl.BlockSpec((TC, HALF, BB), lambda c, t: (t, 0, c)),
            pl.BlockSpec((TC, HALF, BB), lambda c, t: (t, 0, c)),
            pl.BlockSpec((TC, 1, BB), lambda c, t: (t, 0, c)),
            cspec((V, 2 * HALF)), cspec((1, V)),
        ],
        out_specs=pl.BlockSpec((BB, V), lambda c, t: (c, 0)),
        out_shape=jax.ShapeDtypeStruct((B, V), f32),
        scratch_shapes=[
            pltpu.VMEM((HALF, HALF, BB), f32),
            pltpu.VMEM((HALF, HALF, BB), f32),
        ],
        compiler_params=pltpu.CompilerParams(
            dimension_semantics=("parallel", "arbitrary"),
        ),
        name="router_scan",
    )(ksT, keT, gT, jnp.transpose(out_w), out_b.reshape(1, V))
    return out


# speedup vs baseline: 43.1552x; 1.0004x over previous
import functools

import jax
import jax.numpy as jnp
from jax.experimental import pallas as pl
from jax.experimental.pallas import tpu as pltpu

H = 64
V = 64
HALF = 32
B, L = 256, 2048

TSUB = 32
ENC_N = TSUB * B
TC = 128
NTC = L // TC
P = 4
NB = TC // P


def _encoder_body(seq_ref, embT_ref, w1T_ref, b1_ref, w2T_ref, b2_ref,
                  lng_ref, lnb_ref, semT_ref, semb_ref, epiT_ref, epib_ref,
                  rtrT_ref, rtrb_ref, ks_ref, ke_ref, g_ref):
    seq_row = seq_ref[0]
    iota_v = jax.lax.broadcasted_iota(jnp.int32, (V, ENC_N), 0)
    onehot = jnp.where(iota_v == seq_row, 1.0, 0.0)
    e = jnp.dot(embT_ref[...], onehot, preferred_element_type=jnp.float32)

    a1 = jnp.maximum(
        jnp.dot(w1T_ref[...], e, preferred_element_type=jnp.float32)
        + b1_ref[...], 0.0)
    f = jnp.dot(w2T_ref[...], a1, preferred_element_type=jnp.float32) + b2_ref[...]
    x = e + f
    mu = jnp.mean(x, axis=0, keepdims=True)
    xc = x - mu
    var = jnp.mean(xc * xc, axis=0, keepdims=True)
    h = xc * jax.lax.rsqrt(var + 1e-5) * lng_ref[...] + lnb_ref[...]

    ks = jnp.dot(semT_ref[...], h, preferred_element_type=jnp.float32) + semb_ref[...]
    ke = jnp.dot(epiT_ref[...], h, preferred_element_type=jnp.float32) + epib_ref[...]
    gr = jax.nn.sigmoid(
        jnp.dot(rtrT_ref[...], h, preferred_element_type=jnp.float32) + rtrb_ref[...])

    for s in range(TSUB):
        sl = slice(s * B, (s + 1) * B)
        ks_ref[s] = ks[:, sl]
        ke_ref[s] = ke[:, sl]
        g_ref[s] = gr[:, sl]


def _scan_body(ks_ref, ke_ref, g_ref, outwT_ref, outb_ref, out_ref,
               ms_ref, me_ref):
    t = pl.program_id(1)

    @pl.when(t == 0)
    def _():
        ms_ref[...] = jnp.zeros_like(ms_ref)
        me_ref[...] = jnp.zeros_like(me_ref)

    t0 = (t * TC).astype(jnp.float32)

    def sumj(x):
        return jnp.sum(x, axis=0, keepdims=True)

    def rows(k):
        return [k[j:j + 1, :] for j in range(HALF)]

    def load_block(ref, b):
        return [ref[b * P + i] for i in range(P)]

    def row_lists(ref, b, n=P):
        return [[ref[b * P + i, j:j + 1, :] for j in range(HALF)]
                for i in range(n)]

    def w0_pass(m_ref, rws):
        accs = [None] * len(rws)
        for j in range(HALF):
            mj = m_ref[j]
            for tt in range(len(rws)):
                p = mj * rws[tt][j]
                accs[tt] = p if j == 0 else accs[tt] + p
        return accs

    def gates(b, n):
        a_s, a_e = [], []
        for i in range(P):
            if i >= n:
                break
            gg = g_ref[b * P + i]
            rr = (t0 + (b * P + i + 1) * 1.0) * (1.0 / L)
            a_s.append(gg)
            a_e.append((1.0 - gg) * rr)
        return a_s, a_e

    def make_us(kb, w0, a_l, n):
        nn = [sumj(kb[i] * kb[i]) + 1e-6 for i in range(n)]
        d = {}
        for ss in range(n):
            for tt in range(ss + 1, n):
                d[(ss, tt)] = sumj(kb[ss] * kb[tt])
        us = []
        for tt in range(n):
            w = w0[tt]
            for ss in range(tt):
                w = w + d[(ss, tt)] * us[ss]
            ci = a_l[tt] / nn[tt]
            us.append(a_l[tt] * kb[tt] - ci * w)
        return us

    def upd_pass(m_ref, rws, us, nrws):
        accs = [None] * P if nrws is not None else None
        for j in range(HALF):
            mj = m_ref[j]
            for tt in range(len(us)):
                mj = mj + us[tt] * rws[tt][j]
            m_ref[j] = mj
            if nrws is not None:
                for tt in range(P):
                    p = mj * nrws[tt][j]
                    accs[tt] = p if j == 0 else accs[tt] + p
        return accs

    w0s = w0_pass(ms_ref, row_lists(ks_ref, 0))
    w0e = w0_pass(me_ref, row_lists(ke_ref, 0))

    def body(b, carry):
        w0s, w0e = list(carry[:P]), list(carry[P:])
        a_s, a_e = gates(b, P)
        kb = load_block(ks_ref, b)
        us = make_us(kb, w0s, a_s, P)
        nw0s = upd_pass(ms_ref, row_lists(ks_ref, b), us,
                        row_lists(ks_ref, b + 1))
        qb = load_block(ke_ref, b)
        ue = make_us(qb, w0e, a_e, P)
        nw0e = upd_pass(me_ref, row_lists(ke_ref, b), ue,
                        row_lists(ke_ref, b + 1))
        return tuple(nw0s) + tuple(nw0e)

    carry = jax.lax.fori_loop(0, NB - 1, body, tuple(w0s) + tuple(w0e),
                              unroll=False)
    w0s, w0e = list(carry[:P]), list(carry[P:])

    bl = NB - 1
    kb = load_block(ks_ref, bl)
    qb = load_block(ke_ref, bl)

    @pl.when(t < NTC - 1)
    def _():
        a_s, a_e = gates(bl, P)
        us = make_us(kb, w0s, a_s, P)
        upd_pass(ms_ref, row_lists(ks_ref, bl), us, None)
        ue = make_us(qb, w0e, a_e, P)
        upd_pass(me_ref, row_lists(ke_ref, bl), ue, None)

    @pl.when(t == NTC - 1)
    def _():
        a_s, a_e = gates(bl, P - 1)
        us = make_us(kb, w0s, a_s, P - 1)
        upd_pass(ms_ref, row_lists(ks_ref, bl, P - 1), us, None)
        ue = make_us(qb, w0e, a_e, P - 1)
        upd_pass(me_ref, row_lists(ke_ref, bl, P - 1), ue, None)

        cs = w0_pass(ms_ref, [rows(kb[P - 1])])[0]
        ce = w0_pass(me_ref, [rows(qb[P - 1])])[0]
        cat = jnp.concatenate([cs, ce], axis=0)
        outT = jnp.dot(outwT_ref[...], cat, preferred_element_type=jnp.float32)
        out_ref[...] = outT.T + outb_ref[...]


def kernel(seq, embed, ff_w1, ff_b1, ff_w2, ff_b2, ln_g, ln_b,
           sem_w, sem_b, epi_w, epi_b, rtr_w, rtr_b, out_w, out_b):
    f32 = jnp.float32
    seq3 = jnp.transpose(seq).reshape(L // TSUB, 1, ENC_N)
    col = lambda v: v.reshape(-1, 1).astype(f32)

    n_enc = L // TSUB
    wspec = lambda shape: pl.BlockSpec(shape, lambda c, i: (0, 0))
    ksT, keT, gT = pl.pallas_call(
        _encoder_body,
        grid=(2, n_enc // 2),
        in_specs=[
            pl.BlockSpec((1, 1, ENC_N), lambda c, i: (c * (n_enc // 2) + i, 0, 0)),
            wspec((H, V)), wspec((2 * H, H)), wspec((2 * H, 1)),
            wspec((H, 2 * H)), wspec((H, 1)), wspec((H, 1)), wspec((H, 1)),
            wspec((HALF, H)), wspec((HALF, 1)), wspec((HALF, H)), wspec((HALF, 1)),
            wspec((1, H)), wspec((1, 1)),
        ],
        out_specs=[
            pl.BlockSpec((TSUB, HALF, B), lambda c, i: (c * (n_enc // 2) + i, 0, 0)),
            pl.BlockSpec((TSUB, HALF, B), lambda c, i: (c * (n_enc // 2) + i, 0, 0)),
            pl.BlockSpec((TSUB, 1, B), lambda c, i: (c * (n_enc // 2) + i, 0, 0)),
        ],
        out_shape=[
            jax.ShapeDtypeStruct((L, HALF, B), f32),
            jax.ShapeDtypeStruct((L, HALF, B), f32),
            jax.ShapeDtypeStruct((L, 1, B), f32),
        ],
        compiler_params=pltpu.CompilerParams(
            dimension_semantics=("parallel", "arbitrary"),
        ),
        name="router_encoder",
    )(seq3, jnp.transpose(embed), jnp.transpose(ff_w1), col(ff_b1),
      jnp.transpose(ff_w2), col(ff_b2), col(ln_g), col(ln_b),
      jnp.transpose(sem_w), col(sem_b), jnp.transpose(epi_w), col(epi_b),
      jnp.transpose(rtr_w), rtr_b.reshape(1, 1))

    BB = B // 2
    cspec = lambda shape: pl.BlockSpec(shape, lambda c, t: (0, 0))
    out = pl.pallas_call(
        _scan_body,
        grid=(2, NTC),
        in_specs=[
            pl.BlockSpec((TC, HALF, BB), lambda c, t: (t, 0, c)),
            pl.BlockSpec((TC, HALF, BB), lambda c, t: (t, 0, c)),
            pl.BlockSpec((TC, 1, BB), lambda c, t: (t, 0, c)),
            cspec((V, 2 * HALF)), cspec((1, V)),
        ],
        out_specs=pl.BlockSpec((BB, V), lambda c, t: (c, 0)),
        out_shape=jax.ShapeDtypeStruct((B, V), f32),
        scratch_shapes=[
            pltpu.VMEM((HALF, HALF, BB), f32),
            pltpu.VMEM((HALF, HALF, BB), f32),
        ],
        compiler_params=pltpu.CompilerParams(
            dimension_semantics=("parallel", "arbitrary"),
        ),
        name="router_scan",
    )(ksT, keT, gT, jnp.transpose(out_w), out_b.reshape(1, V))
    return out
